# baseline (speedup 1.0000x reference)
import functools

import jax
import jax.numpy as jnp
from jax import lax
from jax.experimental import pallas as pl
from jax.experimental.pallas import tpu as pltpu
from jax.experimental.pallas import tpu_sc as plsc

D = 128
NC = 2
NS = 16
NW = NC * NS
CH = 128
NBUF = 2
PF = NBUF - 1


@functools.lru_cache(maxsize=None)
def _make_kernel(rows, cols):
    B = rows * cols
    per_w = B // NW
    n_chunks = per_w // CH
    n_outer = n_chunks // NBUF
    w_per_row = cols // per_w
    mesh = plsc.VectorSubcoreMesh(core_axis_name="c", subcore_axis_name="s")

    @functools.partial(
        pl.kernel,
        mesh=mesh,
        out_type=(
            jax.ShapeDtypeStruct((B, D), jnp.float32),
            jax.ShapeDtypeStruct((B, D), jnp.float32),
        ),
        scratch_types=[
            pltpu.VMEM((per_w,), jnp.int32),
            pltpu.VMEM((NBUF, CH, D), jnp.float32),
            pltpu.VMEM((NBUF, CH, D), jnp.float32),
        ]
        + [pltpu.SemaphoreType.DMA] * (4 * NBUF),
    )
    def body(pos_hbm, cos_hbm, sin_hbm, cos_out, sin_out,
             idx_v, cbuf, sbuf, *sems):
        gcs = sems[0:NBUF]
        gss = sems[NBUF:2 * NBUF]
        ocs = sems[2 * NBUF:3 * NBUF]
        oss = sems[3 * NBUF:4 * NBUF]
        wid = lax.axis_index("s") * NC + lax.axis_index("c")
        prow = wid // w_per_row
        pcol = (wid % w_per_row) * per_w
        pltpu.sync_copy(pos_hbm.at[prow, pl.ds(pcol, per_w)], idx_v)

        def _gather_pair(c, b):
            ix = idx_v.at[pl.ds(c * CH, CH)]
            return ((cos_hbm.at[ix], cbuf.at[b], gcs[b]),
                    (sin_hbm.at[ix], sbuf.at[b], gss[b]))

        def _out_pair(c, b):
            base = wid * per_w + c * CH
            return ((cbuf.at[b], cos_out.at[pl.ds(base, CH)], ocs[b]),
                    (sbuf.at[b], sin_out.at[pl.ds(base, CH)], oss[b]))

        def gather_issue(c, b):
            for args in _gather_pair(c, b):
                pltpu.async_copy(*args)

        def gather_wait(c, b):
            for args in _gather_pair(c, b):
                pltpu.make_async_copy(*args).wait()

        def out_issue(c, b):
            for args in _out_pair(c, b):
                pltpu.async_copy(*args)

        def out_wait(c, b):
            for args in _out_pair(c, b):
                pltpu.make_async_copy(*args).wait()

        for c0 in range(PF):
            gather_issue(c0, c0)

        def step(g, carry):
            for b in range(NBUF):
                c = g * NBUF + b
                b2 = (b + PF) % NBUF
                gather_wait(c, b)
                if b == 0:
                    @pl.when(g >= 1)
                    def _():
                        out_wait(c - 1, b2)
                else:
                    out_wait(c - 1, b2)
                if b == 0:
                    gather_issue(c + PF, b2)
                else:
                    @pl.when(g < n_outer - 1)
                    def _():
                        gather_issue(c + PF, b2)
                out_issue(c, b)
            return carry

        lax.fori_loop(0, n_outer, step, None)
        out_wait(n_chunks - 1, (n_chunks - 1) % NBUF)

    return body


def kernel(positions, cos, sin):
    pos = positions.astype(jnp.int32)
    cos_out, sin_out = _make_kernel(pos.shape[0], pos.shape[1])(pos, cos, sin)
    return (cos_out, sin_out)


# speedup vs baseline: 1.0370x; 1.0370x over previous
import functools

import jax
import jax.numpy as jnp
from jax import lax
from jax.experimental import pallas as pl
from jax.experimental.pallas import tpu as pltpu
from jax.experimental.pallas import tpu_sc as plsc

D = 128
NC = 2
NS = 16
NW = NC * NS
CH = 64
NBUF = 4
PF = NBUF - 1


@functools.lru_cache(maxsize=None)
def _make_kernel(rows, cols):
    B = rows * cols
    per_w = B // NW
    n_chunks = per_w // CH
    n_outer = n_chunks // NBUF
    w_per_row = cols // per_w
    mesh = plsc.VectorSubcoreMesh(core_axis_name="c", subcore_axis_name="s")

    @functools.partial(
        pl.kernel,
        mesh=mesh,
        out_type=(
            jax.ShapeDtypeStruct((B, D), jnp.float32),
            jax.ShapeDtypeStruct((B, D), jnp.float32),
        ),
        scratch_types=[
            pltpu.VMEM((per_w,), jnp.int32),
            pltpu.VMEM((NBUF, CH, D), jnp.float32),
            pltpu.VMEM((NBUF, CH, D), jnp.float32),
        ]
        + [pltpu.SemaphoreType.DMA] * (4 * NBUF),
    )
    def body(pos_hbm, cos_hbm, sin_hbm, cos_out, sin_out,
             idx_v, cbuf, sbuf, *sems):
        gcs = sems[0:NBUF]
        gss = sems[NBUF:2 * NBUF]
        ocs = sems[2 * NBUF:3 * NBUF]
        oss = sems[3 * NBUF:4 * NBUF]
        wid = lax.axis_index("s") * NC + lax.axis_index("c")
        prow = wid // w_per_row
        pcol = (wid % w_per_row) * per_w
        pltpu.sync_copy(pos_hbm.at[prow, pl.ds(pcol, per_w)], idx_v)

        def _gather_pair(c, b):
            ix = idx_v.at[pl.ds(c * CH, CH)]
            return ((cos_hbm.at[ix], cbuf.at[b], gcs[b]),
                    (sin_hbm.at[ix], sbuf.at[b], gss[b]))

        def _out_pair(c, b):
            base = wid * per_w + c * CH
            return ((cbuf.at[b], cos_out.at[pl.ds(base, CH)], ocs[b]),
                    (sbuf.at[b], sin_out.at[pl.ds(base, CH)], oss[b]))

        def gather_issue(c, b):
            for args in _gather_pair(c, b):
                pltpu.async_copy(*args)

        def gather_wait(c, b):
            for args in _gather_pair(c, b):
                pltpu.make_async_copy(*args).wait()

        def out_issue(c, b):
            for args in _out_pair(c, b):
                pltpu.async_copy(*args)

        def out_wait(c, b):
            for args in _out_pair(c, b):
                pltpu.make_async_copy(*args).wait()

        for c0 in range(PF):
            gather_issue(c0, c0)

        def step(g, carry):
            for b in range(NBUF):
                c = g * NBUF + b
                b2 = (b + PF) % NBUF
                gather_wait(c, b)
                if b == 0:
                    @pl.when(g >= 1)
                    def _():
                        out_wait(c - 1, b2)
                else:
                    out_wait(c - 1, b2)
                if b == 0:
                    gather_issue(c + PF, b2)
                else:
                    @pl.when(g < n_outer - 1)
                    def _():
                        gather_issue(c + PF, b2)
                out_issue(c, b)
            return carry

        lax.fori_loop(0, n_outer, step, None)
        out_wait(n_chunks - 1, (n_chunks - 1) % NBUF)

    return body


def kernel(positions, cos, sin):
    pos = positions.astype(jnp.int32)
    cos_out, sin_out = _make_kernel(pos.shape[0], pos.shape[1])(pos, cos, sin)
    return (cos_out, sin_out)
